# baseline (speedup 1.0000x reference)
import functools

import jax
import jax.numpy as jnp
from jax import lax
from jax.experimental import pallas as pl
from jax.experimental.pallas import tpu as pltpu
from jax.experimental.pallas import tpu_sc as plsc

N = 10000
E = 160000
D = 300
H = 600
L = 5
G = 128

DP = 304
HP = 640
NH = N // 2
TRASH = NH
CH = 48
TILES = 16
CHUNKS = 210
EP = TILES * CHUNKS * CH
SLOTS = CHUNKS * CH + CH
IN_CH = 96
BR = 1000

f32 = jnp.float32
i32 = jnp.int32

_mesh = plsc.VectorSubcoreMesh(core_axis_name="c", subcore_axis_name="s")
_sc_params = pltpu.CompilerParams(use_tc_tiling_on_sc=False,
                                  needs_layout_passes=False)


def _acc_init(z_hbm, acc, s):
    @pl.when(s < TILES - 1)
    def _():
        pltpu.sync_copy(z_hbm.at[pl.ds(s * 320, 320)], acc.at[pl.ds(s * 320, 320)])

    @pl.when(s == TILES - 1)
    def _():
        pltpu.sync_copy(z_hbm.at[pl.ds(4800, 201)], acc.at[pl.ds(4800, 201)])


def _acc_writeback(acc, out_hbm, c, s):
    @pl.when(s < TILES - 1)
    def _():
        pltpu.sync_copy(acc.at[pl.ds(s * 320, 320)],
                        out_hbm.at[pl.ds(c * NH + s * 320, 320)])

    @pl.when(s == TILES - 1)
    def _():
        pltpu.sync_copy(acc.at[pl.ds(4800, 200)],
                        out_hbm.at[pl.ds(c * NH + 4800, 200)])


def _compact_body(src_hbm, dst_hbm, f0_hbm, f1_hbm,
                  csrc_hbm, cdstl_hbm, ccidx_hbm, ccnt_hbm,
                  src_v, dst_v, f0_v, f1_v, st_src, st_dstl, st_cidx, cnt_v):
    c = lax.axis_index("c")
    s = lax.axis_index("s")

    zero16i = jnp.zeros((16,), i32)
    trash16 = jnp.full((16,), TRASH, i32)

    def zinit(j, carry):
        st_src[pl.ds(j * 16, 16)] = zero16i
        st_dstl[pl.ds(j * 16, 16)] = trash16
        st_cidx[pl.ds(j * 16, 16)] = zero16i
        return carry

    lax.fori_loop(0, SLOTS // 16, zinit, 0)

    base0 = s * (CHUNKS * CH)
    off = c * NH

    def scan_step(i, F):
        base = base0 + i * IN_CH
        pltpu.sync_copy(src_hbm.at[pl.ds(base, IN_CH)], src_v)
        pltpu.sync_copy(dst_hbm.at[pl.ds(base, IN_CH)], dst_v)
        pltpu.sync_copy(f0_hbm.at[pl.ds(base, IN_CH)], f0_v)
        pltpu.sync_copy(f1_hbm.at[pl.ds(base, IN_CH)], f1_v)
        for j in range(IN_CH // 16):
            d = dst_v[pl.ds(j * 16, 16)]
            sv = src_v[pl.ds(j * 16, 16)]
            cv = f0_v[pl.ds(j * 16, 16)] * 3 + f1_v[pl.ds(j * 16, 16)]
            loc = d - off
            m = (loc >= 0) & (loc < NH)
            plsc.store_compressed(st_src.at[pl.ds(F, 16)], sv, mask=m)
            plsc.store_compressed(st_dstl.at[pl.ds(F, 16)], loc, mask=m)
            plsc.store_compressed(st_cidx.at[pl.ds(F, 16)], cv, mask=m)
            F = F + jnp.sum(m.astype(i32))
        return F

    F = lax.fori_loop(0, (CHUNKS * CH) // IN_CH, scan_step, jnp.int32(0))
    nchunks = lax.div(F + CH - 1, CH)
    cnt_v[...] = zero16i + nchunks
    pltpu.sync_copy(cnt_v, ccnt_hbm.at[c, s])
    pltpu.sync_copy(st_src, csrc_hbm.at[c, s])
    pltpu.sync_copy(st_dstl, cdstl_hbm.at[c, s])
    pltpu.sync_copy(st_cidx, ccidx_hbm.at[c, s])


_compact = functools.partial(
    pl.kernel,
    out_type=(jax.ShapeDtypeStruct((2, TILES, SLOTS), i32),
              jax.ShapeDtypeStruct((2, TILES, SLOTS), i32),
              jax.ShapeDtypeStruct((2, TILES, SLOTS), i32),
              jax.ShapeDtypeStruct((2, TILES, 16), i32)),
    mesh=_mesh,
    scratch_types=[
        pltpu.VMEM((IN_CH,), i32),
        pltpu.VMEM((IN_CH,), i32),
        pltpu.VMEM((IN_CH,), i32),
        pltpu.VMEM((IN_CH,), i32),
        pltpu.VMEM((SLOTS,), i32),
        pltpu.VMEM((SLOTS,), i32),
        pltpu.VMEM((SLOTS,), i32),
        pltpu.VMEM((16,), i32),
    ],
    compiler_params=_sc_params,
)(_compact_body)


def _spmm_body(h_hbm, csrc_hbm, cdstl_hbm, ccnt_hbm, z_hbm, out_hbm, acc,
               cnt_v,
               src_v0, dstl_v0, rows_v0,
               src_v1, dstl_v1, rows_v1,
               sem_i0, sem_g0, sem_s0, sem_i1, sem_g1, sem_s1):
    c = lax.axis_index("c")
    s = lax.axis_index("s")
    _acc_init(z_hbm, acc, s)
    pltpu.sync_copy(ccnt_hbm.at[c, s], cnt_v)
    nch = jnp.max(cnt_v[...])
    nch2 = jnp.maximum(nch, 2)
    pairs = lax.div(nch2 + 1, 2)
    nch2e = pairs * 2
    plsc.subcore_barrier()

    bufs = ((src_v0, dstl_v0, rows_v0, sem_i0, sem_g0, sem_s0),
            (src_v1, dstl_v1, rows_v1, sem_i1, sem_g1, sem_s1))

    def issue_idx(i, b):
        base = i * CH
        pltpu.async_copy(csrc_hbm.at[c, s, pl.ds(base, CH)], b[0], b[3])
        pltpu.async_copy(cdstl_hbm.at[c, s, pl.ds(base, CH)], b[1], b[3])

    def wait_idx(b):
        pltpu.make_async_copy(csrc_hbm.at[c, s, pl.ds(0, CH)], b[0], b[3]).wait()
        pltpu.make_async_copy(cdstl_hbm.at[c, s, pl.ds(0, CH)], b[1], b[3]).wait()

    def issue_gather(b):
        pltpu.async_copy(h_hbm.at[b[0]], b[2], b[4])

    def wait_gather(b):
        pltpu.make_async_copy(h_hbm.at[b[0]], b[2], b[4]).wait()

    def issue_scatter(b):
        pltpu.async_copy(b[2], acc.at[b[1]], b[5], add=True)

    def wait_scatter(b):
        pltpu.make_async_copy(b[2], acc.at[b[1]], b[5]).wait()

    issue_idx(0, bufs[0])
    issue_idx(1, bufs[1])
    wait_idx(bufs[0])
    issue_gather(bufs[0])

    def pair(k, carry):
        for p in range(2):
            i = k * 2 + p
            cur, oth = bufs[p], bufs[1 - p]

            @pl.when(i + 1 < nch2e)
            def _():
                wait_idx(oth)

                @pl.when(i >= 1)
                def _():
                    wait_scatter(oth)

                issue_gather(oth)

            wait_gather(cur)
            issue_scatter(cur)

            @pl.when(i + 2 < nch2e)
            def _():
                issue_idx(i + 2, cur)
        return carry

    lax.fori_loop(0, pairs, pair, 0)
    wait_scatter(bufs[0])
    wait_scatter(bufs[1])
    plsc.subcore_barrier()
    _acc_writeback(acc, out_hbm, c, s)


def _make_seg_sum(width):
    return functools.partial(
        pl.kernel,
        out_type=jax.ShapeDtypeStruct((N, width), f32),
        mesh=_mesh,
        scratch_types=[
            pltpu.VMEM_SHARED((NH + 1, width), f32),
            pltpu.VMEM((16,), i32),
            pltpu.VMEM((CH,), i32),
            pltpu.VMEM((CH,), i32),
            pltpu.VMEM((CH, width), f32),
            pltpu.VMEM((CH,), i32),
            pltpu.VMEM((CH,), i32),
            pltpu.VMEM((CH, width), f32),
            pltpu.SemaphoreType.DMA,
            pltpu.SemaphoreType.DMA,
            pltpu.SemaphoreType.DMA,
            pltpu.SemaphoreType.DMA,
            pltpu.SemaphoreType.DMA,
            pltpu.SemaphoreType.DMA,
        ],
        compiler_params=_sc_params,
    )(_spmm_body)


_spmm = _make_seg_sum(DP)
_csum = _make_seg_sum(16)


def _embed_body(nf0_ref, nf1_ref, a0_ref, a1_ref, out_ref):
    nf0 = nf0_ref[...]
    nf1 = nf1_ref[...]
    oh0 = (lax.broadcasted_iota(i32, (BR, 128), 1) == nf0).astype(f32)
    oh1 = (lax.broadcasted_iota(i32, (BR, 8), 1) == nf1).astype(f32)
    out_ref[...] = (jnp.dot(oh0, a0_ref[...], preferred_element_type=f32)
                    + jnp.dot(oh1, a1_ref[...], preferred_element_type=f32))


_embed = pl.pallas_call(
    _embed_body,
    grid=(N // BR,),
    in_specs=[
        pl.BlockSpec((BR, 1), lambda i: (i, 0)),
        pl.BlockSpec((BR, 1), lambda i: (i, 0)),
        pl.BlockSpec((128, DP), lambda i: (0, 0)),
        pl.BlockSpec((8, DP), lambda i: (0, 0)),
    ],
    out_specs=pl.BlockSpec((BR, DP), lambda i: (i, 0)),
    out_shape=jax.ShapeDtypeStruct((N, DP), f32),
)


def _mlp_body(relu_out, h_ref, agg_ref, c_ref, ee_ref, w1_ref, b1_ref,
              w2_ref, b2_ref, out_ref):
    z = (h_ref[...] + agg_ref[...]
         + jnp.dot(c_ref[...], ee_ref[...], preferred_element_type=f32))
    a = jnp.maximum(jnp.dot(z, w1_ref[...], preferred_element_type=f32)
                    + b1_ref[...], 0.0)
    y = jnp.dot(a, w2_ref[...], preferred_element_type=f32) + b2_ref[...]
    if relu_out:
        y = jnp.maximum(y, 0.0)
    out_ref[...] = y


def _make_mlp(relu_out):
    return pl.pallas_call(
        functools.partial(_mlp_body, relu_out),
        grid=(N // BR,),
        in_specs=[
            pl.BlockSpec((BR, DP), lambda i: (i, 0)),
            pl.BlockSpec((BR, DP), lambda i: (i, 0)),
            pl.BlockSpec((BR, 16), lambda i: (i, 0)),
            pl.BlockSpec((16, DP), lambda i: (0, 0)),
            pl.BlockSpec((DP, HP), lambda i: (0, 0)),
            pl.BlockSpec((1, HP), lambda i: (0, 0)),
            pl.BlockSpec((HP, DP), lambda i: (0, 0)),
            pl.BlockSpec((1, DP), lambda i: (0, 0)),
        ],
        out_specs=pl.BlockSpec((BR, DP), lambda i: (i, 0)),
        out_shape=jax.ShapeDtypeStruct((N, DP), f32),
    )


_mlp_mid = _make_mlp(True)
_mlp_last = _make_mlp(False)


def _pool_body(h_ref, gid_ref, out_ref, ssum, cnt):
    k = pl.program_id(0)

    @pl.when(k == 0)
    def _():
        ssum[...] = jnp.zeros_like(ssum)
        cnt[...] = jnp.zeros_like(cnt)

    gid = gid_ref[...]
    oh = (lax.broadcasted_iota(i32, (BR, G), 1) == gid).astype(f32)
    ssum[...] += lax.dot_general(oh, h_ref[...], (((0,), (0,)), ((), ())),
                                 preferred_element_type=f32)
    cnt[...] += lax.dot_general(oh, jnp.ones((BR, 128), f32),
                                (((0,), (0,)), ((), ())),
                                preferred_element_type=f32)

    c = cnt[:, 0:1]
    gh = jnp.where(c > 0, ssum[...] / jnp.maximum(c, 1.0), 0.0)
    out_ref[...] = jnp.maximum(gh[:, :D], 0.0)


_pool = pl.pallas_call(
    _pool_body,
    grid=(N // BR,),
    in_specs=[
        pl.BlockSpec((BR, DP), lambda i: (i, 0)),
        pl.BlockSpec((BR, 1), lambda i: (i, 0)),
    ],
    out_specs=pl.BlockSpec((G, D), lambda i: (0, 0)),
    out_shape=jax.ShapeDtypeStruct((G, D), f32),
    scratch_shapes=[pltpu.VMEM((G, DP), f32), pltpu.VMEM((G, 128), f32)],
)


def kernel(atom_emb0, atom_emb1, edge_emb0, edge_emb1, W1, b1, W2, b2,
           gamma, beta, rmean, rvar,
           node_feat0, node_feat1, edge_index, edge_feat0, edge_feat1,
           graph_ids):
    bn_s = gamma / jnp.sqrt(rvar + 1e-5)
    W2f = W2 * bn_s[:, None, :]
    b2f = b2 * bn_s + (beta - rmean * bn_s)

    W1p = jnp.zeros((L, DP, HP), f32).at[:, :D, :H].set(W1)
    b1p = jnp.zeros((L, 1, HP), f32).at[:, 0, :H].set(b1)
    W2p = jnp.zeros((L, HP, DP), f32).at[:, :H, :D].set(W2f)
    b2p = jnp.zeros((L, 1, DP), f32).at[:, 0, :D].set(b2f)
    EE = (jnp.zeros((L, 16, DP), f32)
          .at[:, 0:6, :D].set(edge_emb0)
          .at[:, 6:9, :D].set(edge_emb1))
    A0p = jnp.zeros((128, DP), f32).at[:120, :D].set(atom_emb0)
    A1p = jnp.zeros((8, DP), f32).at[:3, :D].set(atom_emb1)

    nf0 = node_feat0.astype(i32).reshape(N, 1)
    nf1 = node_feat1.astype(i32).reshape(N, 1)
    src = edge_index[0].astype(i32)
    dst = edge_index[1].astype(i32)
    pad = EP - E
    srcp = jnp.concatenate([src, jnp.zeros((pad,), i32)])
    dstp = jnp.concatenate([dst, jnp.full((pad,), N, i32)])
    f0p = jnp.concatenate([edge_feat0.astype(i32), jnp.zeros((pad,), i32)])
    f1p = jnp.concatenate([edge_feat1.astype(i32), jnp.zeros((pad,), i32)])
    gid = graph_ids.astype(i32).reshape(N, 1)
    Z = jnp.zeros((NH + 1, DP), f32)
    Z16 = jnp.zeros((NH + 1, 16), f32)

    r18 = jnp.arange(18)
    Tp = (jnp.zeros((24, 16), f32)
          .at[r18, r18 // 3].add(1.0)
          .at[r18, r18 % 3 + 6].add(1.0))

    h = _embed(nf0, nf1, A0p, A1p)
    csrc, cdstl, ccidx, ccnt = _compact(srcp, dstp, f0p, f1p)
    C = _csum(Tp, ccidx, cdstl, ccnt, Z16)
    for l in range(L):
        agg = _spmm(h, csrc, cdstl, ccnt, Z)
        mlp = _mlp_mid if l < L - 1 else _mlp_last
        h = mlp(h, agg, C, EE[l], W1p[l], b1p[l], W2p[l], b2p[l])
    return _pool(h, gid)


# speedup vs baseline: 1.0918x; 1.0918x over previous
import functools

import jax
import jax.numpy as jnp
from jax import lax
from jax.experimental import pallas as pl
from jax.experimental.pallas import tpu as pltpu
from jax.experimental.pallas import tpu_sc as plsc

N = 10000
E = 160000
D = 300
H = 600
L = 5
G = 128

DP = 304
HP = 640
NH = N // 2
TRASH = NH
CH = 48
TILES = 16
CHUNKS = 210
EP = TILES * CHUNKS * CH
SLOTS = CHUNKS * CH + CH
IN_CH = 720
IN_STEPS = (CHUNKS * CH) // IN_CH
BR = 1000

f32 = jnp.float32
i32 = jnp.int32

_mesh = plsc.VectorSubcoreMesh(core_axis_name="c", subcore_axis_name="s")
_sc_params = pltpu.CompilerParams(use_tc_tiling_on_sc=False,
                                  needs_layout_passes=False)


def _acc_init(z_hbm, acc, s):
    @pl.when(s < TILES - 1)
    def _():
        pltpu.sync_copy(z_hbm.at[pl.ds(s * 320, 320)], acc.at[pl.ds(s * 320, 320)])

    @pl.when(s == TILES - 1)
    def _():
        pltpu.sync_copy(z_hbm.at[pl.ds(4800, 201)], acc.at[pl.ds(4800, 201)])


def _acc_writeback(acc, out_hbm, c, s):
    @pl.when(s < TILES - 1)
    def _():
        pltpu.sync_copy(acc.at[pl.ds(s * 320, 320)],
                        out_hbm.at[pl.ds(c * NH + s * 320, 320)])

    @pl.when(s == TILES - 1)
    def _():
        pltpu.sync_copy(acc.at[pl.ds(4800, 200)],
                        out_hbm.at[pl.ds(c * NH + 4800, 200)])


def _compact_body(src_hbm, dst_hbm, f0_hbm, f1_hbm,
                  csrc_hbm, cdstl_hbm, ccidx_hbm, ccnt_hbm,
                  src_v0, dst_v0, f0_v0, f1_v0,
                  src_v1, dst_v1, f0_v1, f1_v1,
                  st_src, st_dstl, st_cidx, cnt_v, sem0, sem1):
    c = lax.axis_index("c")
    s = lax.axis_index("s")

    zero16i = jnp.zeros((16,), i32)
    trash16 = jnp.full((16,), TRASH, i32)

    def zinit(j, carry):
        st_src[pl.ds(j * 16, 16)] = zero16i
        st_dstl[pl.ds(j * 16, 16)] = trash16
        st_cidx[pl.ds(j * 16, 16)] = zero16i
        return carry

    lax.fori_loop(0, SLOTS // 16, zinit, 0)

    base0 = s * (CHUNKS * CH)
    off = c * NH
    bufs = ((src_v0, dst_v0, f0_v0, f1_v0, sem0),
            (src_v1, dst_v1, f0_v1, f1_v1, sem1))
    srcs = (src_hbm, dst_hbm, f0_hbm, f1_hbm)

    def issue_in(i, b):
        base = base0 + i * IN_CH
        for k in range(4):
            pltpu.async_copy(srcs[k].at[pl.ds(base, IN_CH)], b[k], b[4])

    def wait_in(b):
        for k in range(4):
            pltpu.make_async_copy(srcs[k].at[pl.ds(base0, IN_CH)],
                                  b[k], b[4]).wait()

    def process(b, F):
        for j in range(IN_CH // 16):
            d = b[1][pl.ds(j * 16, 16)]
            sv = b[0][pl.ds(j * 16, 16)]
            cv = b[2][pl.ds(j * 16, 16)] * 3 + b[3][pl.ds(j * 16, 16)]
            loc = d - off
            m = (loc >= 0) & (loc < NH)
            plsc.store_compressed(st_src.at[pl.ds(F, 16)], sv, mask=m)
            plsc.store_compressed(st_dstl.at[pl.ds(F, 16)], loc, mask=m)
            plsc.store_compressed(st_cidx.at[pl.ds(F, 16)], cv, mask=m)
            F = F + jnp.sum(m.astype(i32))
        return F

    issue_in(0, bufs[0])
    issue_in(1, bufs[1])

    def pair(k, F):
        for p in range(2):
            i = k * 2 + p
            wait_in(bufs[p])
            F = process(bufs[p], F)

            @pl.when(i + 2 < IN_STEPS)
            def _():
                issue_in(i + 2, bufs[p])
        return F

    F = lax.fori_loop(0, IN_STEPS // 2, pair, jnp.int32(0))
    nchunks = lax.div(F + CH - 1, CH)
    cnt_v[...] = zero16i + nchunks
    pltpu.sync_copy(cnt_v, ccnt_hbm.at[c, s])
    pltpu.sync_copy(st_src, csrc_hbm.at[c, s])
    pltpu.sync_copy(st_dstl, cdstl_hbm.at[c, s])
    pltpu.sync_copy(st_cidx, ccidx_hbm.at[c, s])


_compact = functools.partial(
    pl.kernel,
    out_type=(jax.ShapeDtypeStruct((2, TILES, SLOTS), i32),
              jax.ShapeDtypeStruct((2, TILES, SLOTS), i32),
              jax.ShapeDtypeStruct((2, TILES, SLOTS), i32),
              jax.ShapeDtypeStruct((2, TILES, 16), i32)),
    mesh=_mesh,
    scratch_types=[
        pltpu.VMEM((IN_CH,), i32),
        pltpu.VMEM((IN_CH,), i32),
        pltpu.VMEM((IN_CH,), i32),
        pltpu.VMEM((IN_CH,), i32),
        pltpu.VMEM((IN_CH,), i32),
        pltpu.VMEM((IN_CH,), i32),
        pltpu.VMEM((IN_CH,), i32),
        pltpu.VMEM((IN_CH,), i32),
        pltpu.VMEM((SLOTS,), i32),
        pltpu.VMEM((SLOTS,), i32),
        pltpu.VMEM((SLOTS,), i32),
        pltpu.VMEM((16,), i32),
        pltpu.SemaphoreType.DMA,
        pltpu.SemaphoreType.DMA,
    ],
    compiler_params=_sc_params,
)(_compact_body)


def _spmm_body(h_hbm, csrc_hbm, cdstl_hbm, ccnt_hbm, z_hbm, out_hbm, acc,
               cnt_v,
               src_v0, dstl_v0, rows_v0,
               src_v1, dstl_v1, rows_v1,
               sem_i0, sem_g0, sem_s0, sem_i1, sem_g1, sem_s1):
    c = lax.axis_index("c")
    s = lax.axis_index("s")
    _acc_init(z_hbm, acc, s)
    pltpu.sync_copy(ccnt_hbm.at[c, s], cnt_v)
    nch = jnp.max(cnt_v[...])
    nch2 = jnp.maximum(nch, 2)
    pairs = lax.div(nch2 + 1, 2)
    nch2e = pairs * 2
    plsc.subcore_barrier()

    bufs = ((src_v0, dstl_v0, rows_v0, sem_i0, sem_g0, sem_s0),
            (src_v1, dstl_v1, rows_v1, sem_i1, sem_g1, sem_s1))

    def issue_idx(i, b):
        base = i * CH
        pltpu.async_copy(csrc_hbm.at[c, s, pl.ds(base, CH)], b[0], b[3])
        pltpu.async_copy(cdstl_hbm.at[c, s, pl.ds(base, CH)], b[1], b[3])

    def wait_idx(b):
        pltpu.make_async_copy(csrc_hbm.at[c, s, pl.ds(0, CH)], b[0], b[3]).wait()
        pltpu.make_async_copy(cdstl_hbm.at[c, s, pl.ds(0, CH)], b[1], b[3]).wait()

    def issue_gather(b):
        pltpu.async_copy(h_hbm.at[b[0]], b[2], b[4])

    def wait_gather(b):
        pltpu.make_async_copy(h_hbm.at[b[0]], b[2], b[4]).wait()

    def issue_scatter(b):
        pltpu.async_copy(b[2], acc.at[b[1]], b[5], add=True)

    def wait_scatter(b):
        pltpu.make_async_copy(b[2], acc.at[b[1]], b[5]).wait()

    issue_idx(0, bufs[0])
    issue_idx(1, bufs[1])
    wait_idx(bufs[0])
    issue_gather(bufs[0])

    def pair(k, carry):
        for p in range(2):
            i = k * 2 + p
            cur, oth = bufs[p], bufs[1 - p]

            @pl.when(i + 1 < nch2e)
            def _():
                wait_idx(oth)

                @pl.when(i >= 1)
                def _():
                    wait_scatter(oth)

                issue_gather(oth)

            wait_gather(cur)
            issue_scatter(cur)

            @pl.when(i + 2 < nch2e)
            def _():
                issue_idx(i + 2, cur)
        return carry

    lax.fori_loop(0, pairs, pair, 0)
    wait_scatter(bufs[0])
    wait_scatter(bufs[1])
    plsc.subcore_barrier()
    _acc_writeback(acc, out_hbm, c, s)


def _make_seg_sum(width):
    return functools.partial(
        pl.kernel,
        out_type=jax.ShapeDtypeStruct((N, width), f32),
        mesh=_mesh,
        scratch_types=[
            pltpu.VMEM_SHARED((NH + 1, width), f32),
            pltpu.VMEM((16,), i32),
            pltpu.VMEM((CH,), i32),
            pltpu.VMEM((CH,), i32),
            pltpu.VMEM((CH, width), f32),
            pltpu.VMEM((CH,), i32),
            pltpu.VMEM((CH,), i32),
            pltpu.VMEM((CH, width), f32),
            pltpu.SemaphoreType.DMA,
            pltpu.SemaphoreType.DMA,
            pltpu.SemaphoreType.DMA,
            pltpu.SemaphoreType.DMA,
            pltpu.SemaphoreType.DMA,
            pltpu.SemaphoreType.DMA,
        ],
        compiler_params=_sc_params,
    )(_spmm_body)


_spmm = _make_seg_sum(DP)
_csum = _make_seg_sum(16)


def _embed_body(nf0_ref, nf1_ref, a0_ref, a1_ref, out_ref):
    nf0 = nf0_ref[...]
    nf1 = nf1_ref[...]
    oh0 = (lax.broadcasted_iota(i32, (BR, 128), 1) == nf0).astype(f32)
    oh1 = (lax.broadcasted_iota(i32, (BR, 8), 1) == nf1).astype(f32)
    out_ref[...] = (jnp.dot(oh0, a0_ref[...], preferred_element_type=f32)
                    + jnp.dot(oh1, a1_ref[...], preferred_element_type=f32))


_embed = pl.pallas_call(
    _embed_body,
    grid=(N // BR,),
    in_specs=[
        pl.BlockSpec((BR, 1), lambda i: (i, 0)),
        pl.BlockSpec((BR, 1), lambda i: (i, 0)),
        pl.BlockSpec((128, DP), lambda i: (0, 0)),
        pl.BlockSpec((8, DP), lambda i: (0, 0)),
    ],
    out_specs=pl.BlockSpec((BR, DP), lambda i: (i, 0)),
    out_shape=jax.ShapeDtypeStruct((N, DP), f32),
)


def _mlp_body(relu_out, h_ref, agg_ref, c_ref, ee_ref, w1_ref, b1_ref,
              w2_ref, b2_ref, out_ref):
    z = (h_ref[...] + agg_ref[...]
         + jnp.dot(c_ref[...], ee_ref[...], preferred_element_type=f32))
    a = jnp.maximum(jnp.dot(z, w1_ref[...], preferred_element_type=f32)
                    + b1_ref[...], 0.0)
    y = jnp.dot(a, w2_ref[...], preferred_element_type=f32) + b2_ref[...]
    if relu_out:
        y = jnp.maximum(y, 0.0)
    out_ref[...] = y


def _make_mlp(relu_out):
    return pl.pallas_call(
        functools.partial(_mlp_body, relu_out),
        grid=(N // BR,),
        in_specs=[
            pl.BlockSpec((BR, DP), lambda i: (i, 0)),
            pl.BlockSpec((BR, DP), lambda i: (i, 0)),
            pl.BlockSpec((BR, 16), lambda i: (i, 0)),
            pl.BlockSpec((16, DP), lambda i: (0, 0)),
            pl.BlockSpec((DP, HP), lambda i: (0, 0)),
            pl.BlockSpec((1, HP), lambda i: (0, 0)),
            pl.BlockSpec((HP, DP), lambda i: (0, 0)),
            pl.BlockSpec((1, DP), lambda i: (0, 0)),
        ],
        out_specs=pl.BlockSpec((BR, DP), lambda i: (i, 0)),
        out_shape=jax.ShapeDtypeStruct((N, DP), f32),
    )


_mlp_mid = _make_mlp(True)
_mlp_last = _make_mlp(False)


def _pool_body(h_ref, gid_ref, out_ref, ssum, cnt):
    k = pl.program_id(0)

    @pl.when(k == 0)
    def _():
        ssum[...] = jnp.zeros_like(ssum)
        cnt[...] = jnp.zeros_like(cnt)

    gid = gid_ref[...]
    oh = (lax.broadcasted_iota(i32, (BR, G), 1) == gid).astype(f32)
    ssum[...] += lax.dot_general(oh, h_ref[...], (((0,), (0,)), ((), ())),
                                 preferred_element_type=f32)
    cnt[...] += lax.dot_general(oh, jnp.ones((BR, 128), f32),
                                (((0,), (0,)), ((), ())),
                                preferred_element_type=f32)

    c = cnt[:, 0:1]
    gh = jnp.where(c > 0, ssum[...] / jnp.maximum(c, 1.0), 0.0)
    out_ref[...] = jnp.maximum(gh[:, :D], 0.0)


_pool = pl.pallas_call(
    _pool_body,
    grid=(N // BR,),
    in_specs=[
        pl.BlockSpec((BR, DP), lambda i: (i, 0)),
        pl.BlockSpec((BR, 1), lambda i: (i, 0)),
    ],
    out_specs=pl.BlockSpec((G, D), lambda i: (0, 0)),
    out_shape=jax.ShapeDtypeStruct((G, D), f32),
    scratch_shapes=[pltpu.VMEM((G, DP), f32), pltpu.VMEM((G, 128), f32)],
)


def kernel(atom_emb0, atom_emb1, edge_emb0, edge_emb1, W1, b1, W2, b2,
           gamma, beta, rmean, rvar,
           node_feat0, node_feat1, edge_index, edge_feat0, edge_feat1,
           graph_ids):
    bn_s = gamma / jnp.sqrt(rvar + 1e-5)
    W2f = W2 * bn_s[:, None, :]
    b2f = b2 * bn_s + (beta - rmean * bn_s)

    W1p = jnp.zeros((L, DP, HP), f32).at[:, :D, :H].set(W1)
    b1p = jnp.zeros((L, 1, HP), f32).at[:, 0, :H].set(b1)
    W2p = jnp.zeros((L, HP, DP), f32).at[:, :H, :D].set(W2f)
    b2p = jnp.zeros((L, 1, DP), f32).at[:, 0, :D].set(b2f)
    EE = (jnp.zeros((L, 16, DP), f32)
          .at[:, 0:6, :D].set(edge_emb0)
          .at[:, 6:9, :D].set(edge_emb1))
    A0p = jnp.zeros((128, DP), f32).at[:120, :D].set(atom_emb0)
    A1p = jnp.zeros((8, DP), f32).at[:3, :D].set(atom_emb1)

    nf0 = node_feat0.astype(i32).reshape(N, 1)
    nf1 = node_feat1.astype(i32).reshape(N, 1)
    src = edge_index[0].astype(i32)
    dst = edge_index[1].astype(i32)
    pad = EP - E
    srcp = jnp.concatenate([src, jnp.zeros((pad,), i32)])
    dstp = jnp.concatenate([dst, jnp.full((pad,), N, i32)])
    f0p = jnp.concatenate([edge_feat0.astype(i32), jnp.zeros((pad,), i32)])
    f1p = jnp.concatenate([edge_feat1.astype(i32), jnp.zeros((pad,), i32)])
    gid = graph_ids.astype(i32).reshape(N, 1)
    Z = jnp.zeros((NH + 1, DP), f32)
    Z16 = jnp.zeros((NH + 1, 16), f32)

    r18 = jnp.arange(18)
    Tp = (jnp.zeros((24, 16), f32)
          .at[r18, r18 // 3].add(1.0)
          .at[r18, r18 % 3 + 6].add(1.0))

    h = _embed(nf0, nf1, A0p, A1p)
    csrc, cdstl, ccidx, ccnt = _compact(srcp, dstp, f0p, f1p)
    C = _csum(Tp, ccidx, cdstl, ccnt, Z16)
    for l in range(L):
        agg = _spmm(h, csrc, cdstl, ccnt, Z)
        mlp = _mlp_mid if l < L - 1 else _mlp_last
        h = mlp(h, agg, C, EE[l], W1p[l], b1p[l], W2p[l], b2p[l])
    return _pool(h, gid)


# speedup vs baseline: 1.3373x; 1.2248x over previous
import functools

import jax
import jax.numpy as jnp
from jax import lax
from jax.experimental import pallas as pl
from jax.experimental.pallas import tpu as pltpu
from jax.experimental.pallas import tpu_sc as plsc

N = 10000
E = 160000
D = 300
H = 600
L = 5
G = 128

DP = 304
HP = 640
NH = N // 2
TRASH = NH
CH = 48
TILES = 16
CHUNKS = 210
EP = TILES * CHUNKS * CH
SLOTS = 10240
CH2 = 128
IN_CH = 720
IN_STEPS = (CHUNKS * CH) // IN_CH
BR = 1000

f32 = jnp.float32
i32 = jnp.int32

_mesh = plsc.VectorSubcoreMesh(core_axis_name="c", subcore_axis_name="s")
_sc_params = pltpu.CompilerParams(use_tc_tiling_on_sc=False,
                                  needs_layout_passes=False)


def _acc_init(z_hbm, acc, s):
    @pl.when(s < TILES - 1)
    def _():
        pltpu.sync_copy(z_hbm.at[pl.ds(s * 320, 320)], acc.at[pl.ds(s * 320, 320)])

    @pl.when(s == TILES - 1)
    def _():
        pltpu.sync_copy(z_hbm.at[pl.ds(4800, 201)], acc.at[pl.ds(4800, 201)])


def _acc_writeback(acc, out_hbm, c, s):
    @pl.when(s < TILES - 1)
    def _():
        pltpu.sync_copy(acc.at[pl.ds(s * 320, 320)],
                        out_hbm.at[pl.ds(c * NH + s * 320, 320)])

    @pl.when(s == TILES - 1)
    def _():
        pltpu.sync_copy(acc.at[pl.ds(4800, 200)],
                        out_hbm.at[pl.ds(c * NH + 4800, 200)])


def _compact_body(src_hbm, dst_hbm, f0_hbm, f1_hbm,
                  csrc_hbm, cdstl_hbm, ccidx_hbm, ccnt_hbm,
                  src_v0, dst_v0, f0_v0, f1_v0,
                  src_v1, dst_v1, f0_v1, f1_v1,
                  st_src, st_dstl, st_cidx, cnt_v, sem0, sem1):
    c = lax.axis_index("c")
    s = lax.axis_index("s")

    zero16i = jnp.zeros((16,), i32)
    trash16 = jnp.full((16,), TRASH, i32)

    def zinit(j, carry):
        st_src[pl.ds(j * 16, 16)] = zero16i
        st_dstl[pl.ds(j * 16, 16)] = trash16
        st_cidx[pl.ds(j * 16, 16)] = zero16i
        return carry

    lax.fori_loop(0, SLOTS // 16, zinit, 0)

    base0 = s * (CHUNKS * CH)
    off = c * NH
    bufs = ((src_v0, dst_v0, f0_v0, f1_v0, sem0),
            (src_v1, dst_v1, f0_v1, f1_v1, sem1))
    srcs = (src_hbm, dst_hbm, f0_hbm, f1_hbm)

    def issue_in(i, b):
        base = base0 + i * IN_CH
        for k in range(4):
            pltpu.async_copy(srcs[k].at[pl.ds(base, IN_CH)], b[k], b[4])

    def wait_in(b):
        for k in range(4):
            pltpu.make_async_copy(srcs[k].at[pl.ds(base0, IN_CH)],
                                  b[k], b[4]).wait()

    def process(b, F):
        for j in range(IN_CH // 16):
            d = b[1][pl.ds(j * 16, 16)]
            sv = b[0][pl.ds(j * 16, 16)]
            cv = b[2][pl.ds(j * 16, 16)] * 3 + b[3][pl.ds(j * 16, 16)]
            loc = d - off
            m = (loc >= 0) & (loc < NH)
            plsc.store_compressed(st_src.at[pl.ds(F, 16)], sv, mask=m)
            plsc.store_compressed(st_dstl.at[pl.ds(F, 16)], loc, mask=m)
            plsc.store_compressed(st_cidx.at[pl.ds(F, 16)], cv, mask=m)
            F = F + jnp.sum(m.astype(i32))
        return F

    issue_in(0, bufs[0])
    issue_in(1, bufs[1])

    def pair(k, F):
        for p in range(2):
            i = k * 2 + p
            wait_in(bufs[p])
            F = process(bufs[p], F)

            @pl.when(i + 2 < IN_STEPS)
            def _():
                issue_in(i + 2, bufs[p])
        return F

    F = lax.fori_loop(0, IN_STEPS // 2, pair, jnp.int32(0))
    nchunks = lax.div(F + CH - 1, CH)
    cnt_v[...] = zero16i + nchunks
    pltpu.sync_copy(cnt_v, ccnt_hbm.at[c, s])
    pltpu.sync_copy(st_src, csrc_hbm.at[c, s])
    pltpu.sync_copy(st_dstl, cdstl_hbm.at[c, s])
    pltpu.sync_copy(st_cidx, ccidx_hbm.at[c, s])


_compact = functools.partial(
    pl.kernel,
    out_type=(jax.ShapeDtypeStruct((2, TILES, SLOTS), i32),
              jax.ShapeDtypeStruct((2, TILES, SLOTS), i32),
              jax.ShapeDtypeStruct((2, TILES, SLOTS), i32),
              jax.ShapeDtypeStruct((2, TILES, 16), i32)),
    mesh=_mesh,
    scratch_types=[
        pltpu.VMEM((IN_CH,), i32),
        pltpu.VMEM((IN_CH,), i32),
        pltpu.VMEM((IN_CH,), i32),
        pltpu.VMEM((IN_CH,), i32),
        pltpu.VMEM((IN_CH,), i32),
        pltpu.VMEM((IN_CH,), i32),
        pltpu.VMEM((IN_CH,), i32),
        pltpu.VMEM((IN_CH,), i32),
        pltpu.VMEM((SLOTS,), i32),
        pltpu.VMEM((SLOTS,), i32),
        pltpu.VMEM((SLOTS,), i32),
        pltpu.VMEM((16,), i32),
        pltpu.SemaphoreType.DMA,
        pltpu.SemaphoreType.DMA,
    ],
    compiler_params=_sc_params,
)(_compact_body)


def _spmm_body(h_hbm, csrc_hbm, cdstl_hbm, ccnt_hbm, z_hbm, out_hbm, acc,
               cnt_v,
               src_v0, dstl_v0, dstlsc_v0, rows_v0,
               src_v1, dstl_v1, dstlsc_v1, rows_v1,
               sem_i0, sem_g0, sem_s0, sem_i1, sem_g1, sem_s1):
    c = lax.axis_index("c")
    s = lax.axis_index("s")
    _acc_init(z_hbm, acc, s)
    pltpu.sync_copy(ccnt_hbm.at[c, s], cnt_v)
    nch = jnp.max(cnt_v[...])
    nch2 = jnp.maximum(nch, 2)
    pairs = lax.div(nch2 + 1, 2)
    nch2e = pairs * 2
    plsc.subcore_barrier()

    bufs = ((src_v0, dstl_v0, rows_v0, sem_i0, sem_g0, sem_s0, dstlsc_v0),
            (src_v1, dstl_v1, rows_v1, sem_i1, sem_g1, sem_s1, dstlsc_v1))

    def issue_idx(i, b):
        base = i * CH
        pltpu.async_copy(csrc_hbm.at[c, s, pl.ds(base, CH)], b[0], b[3])
        pltpu.async_copy(cdstl_hbm.at[c, s, pl.ds(base, CH)], b[1], b[3])

    def wait_idx(b):
        pltpu.make_async_copy(csrc_hbm.at[c, s, pl.ds(0, CH)], b[0], b[3]).wait()
        pltpu.make_async_copy(cdstl_hbm.at[c, s, pl.ds(0, CH)], b[1], b[3]).wait()

    def issue_gather(b):
        pltpu.async_copy(h_hbm.at[b[0]], b[2], b[4])

    def wait_gather(b):
        pltpu.make_async_copy(h_hbm.at[b[0]], b[2], b[4]).wait()

    def issue_scatter(b):
        for j in range(CH // 16):
            b[6][pl.ds(j * 16, 16)] = b[1][pl.ds(j * 16, 16)]
        pltpu.async_copy(b[2], acc.at[b[6]], b[5], add=True)

    def wait_scatter(b):
        pltpu.make_async_copy(b[2], acc.at[b[6]], b[5]).wait()

    issue_idx(0, bufs[0])
    issue_idx(1, bufs[1])
    wait_idx(bufs[0])
    issue_gather(bufs[0])

    def pair(k, carry):
        for p in range(2):
            i = k * 2 + p
            cur, oth = bufs[p], bufs[1 - p]

            @pl.when(i + 1 < nch2e)
            def _():
                wait_idx(oth)

                @pl.when(i >= 1)
                def _():
                    wait_scatter(oth)

                issue_gather(oth)

            wait_gather(cur)
            issue_scatter(cur)

            @pl.when(i + 2 < nch2e)
            def _():
                issue_idx(i + 2, cur)
        return carry

    lax.fori_loop(0, pairs, pair, 0)
    wait_scatter(bufs[0])
    wait_scatter(bufs[1])
    plsc.subcore_barrier()
    _acc_writeback(acc, out_hbm, c, s)


def _make_seg_sum(width):
    return functools.partial(
        pl.kernel,
        out_type=jax.ShapeDtypeStruct((N, width), f32),
        mesh=_mesh,
        scratch_types=[
            pltpu.VMEM_SHARED((NH + 1, width), f32),
            pltpu.VMEM((16,), i32),
            pltpu.VMEM((CH,), i32),
            pltpu.VMEM((CH,), i32),
            pltpu.VMEM((CH,), i32),
            pltpu.VMEM((CH, width), f32),
            pltpu.VMEM((CH,), i32),
            pltpu.VMEM((CH,), i32),
            pltpu.VMEM((CH,), i32),
            pltpu.VMEM((CH, width), f32),
            pltpu.SemaphoreType.DMA,
            pltpu.SemaphoreType.DMA,
            pltpu.SemaphoreType.DMA,
            pltpu.SemaphoreType.DMA,
            pltpu.SemaphoreType.DMA,
            pltpu.SemaphoreType.DMA,
        ],
        compiler_params=_sc_params,
    )(_spmm_body)


_spmm = _make_seg_sum(DP)


def _csum_body(tab_hbm, ccidx_hbm, cdstl_hbm, ccnt_hbm, z16_hbm, out_hbm,
               acc, tab_v, cnt_v,
               cidx_v0, dstl_v0, dstlsc_v0, rows_v0,
               cidx_v1, dstl_v1, dstlsc_v1, rows_v1,
               sem_i0, sem_s0, sem_i1, sem_s1):
    c = lax.axis_index("c")
    s = lax.axis_index("s")
    _acc_init(z16_hbm, acc, s)
    pltpu.sync_copy(tab_hbm, tab_v)
    pltpu.sync_copy(ccnt_hbm.at[c, s], cnt_v)
    nch48 = jnp.max(cnt_v[...])
    nch = lax.div(nch48 * CH + CH2 - 1, CH2)
    nch2 = jnp.maximum(nch, 2)
    pairs = lax.div(nch2 + 1, 2)
    nch2e = pairs * 2
    plsc.subcore_barrier()

    bufs = ((cidx_v0, dstl_v0, dstlsc_v0, rows_v0, sem_i0, sem_s0),
            (cidx_v1, dstl_v1, dstlsc_v1, rows_v1, sem_i1, sem_s1))
    lanes = lax.iota(i32, 16)

    def issue_idx(i, b):
        base = i * CH2
        pltpu.async_copy(ccidx_hbm.at[c, s, pl.ds(base, CH2)], b[0], b[4])
        pltpu.async_copy(cdstl_hbm.at[c, s, pl.ds(base, CH2)], b[1], b[4])

    def wait_idx(b):
        pltpu.make_async_copy(ccidx_hbm.at[c, s, pl.ds(0, CH2)], b[0], b[4]).wait()
        pltpu.make_async_copy(cdstl_hbm.at[c, s, pl.ds(0, CH2)], b[1], b[4]).wait()

    def construct(b):
        for g in range(CH2 // 16):
            cidx = b[0][pl.ds(g * 16, 16)]
            rowi = lanes + g * 16
            for j in range(16):
                colj = jnp.full((16,), j, i32)
                vals = plsc.load_gather(tab_v, [cidx, colj])
                plsc.store_scatter(b[3], [rowi, colj], vals)
            b[2][pl.ds(g * 16, 16)] = b[1][pl.ds(g * 16, 16)]

    def issue_scatter(b):
        pltpu.async_copy(b[3], acc.at[b[2]], b[5], add=True)

    def wait_scatter(b):
        pltpu.make_async_copy(b[3], acc.at[b[2]], b[5]).wait()

    issue_idx(0, bufs[0])
    issue_idx(1, bufs[1])

    def pair(k, carry):
        for p in range(2):
            i = k * 2 + p
            cur = bufs[p]
            wait_idx(cur)

            @pl.when(i >= 2)
            def _():
                wait_scatter(cur)

            construct(cur)
            issue_scatter(cur)

            @pl.when(i + 2 < nch2e)
            def _():
                issue_idx(i + 2, cur)
        return carry

    lax.fori_loop(0, pairs, pair, 0)
    wait_scatter(bufs[0])
    wait_scatter(bufs[1])
    plsc.subcore_barrier()
    _acc_writeback(acc, out_hbm, c, s)


_csum = functools.partial(
    pl.kernel,
    out_type=jax.ShapeDtypeStruct((N, 16), f32),
    mesh=_mesh,
    scratch_types=[
        pltpu.VMEM_SHARED((NH + 1, 16), f32),
        pltpu.VMEM((24, 16), f32),
        pltpu.VMEM((16,), i32),
        pltpu.VMEM((CH2,), i32),
        pltpu.VMEM((CH2,), i32),
        pltpu.VMEM((CH2,), i32),
        pltpu.VMEM((CH2, 16), f32),
        pltpu.VMEM((CH2,), i32),
        pltpu.VMEM((CH2,), i32),
        pltpu.VMEM((CH2,), i32),
        pltpu.VMEM((CH2, 16), f32),
        pltpu.SemaphoreType.DMA,
        pltpu.SemaphoreType.DMA,
        pltpu.SemaphoreType.DMA,
        pltpu.SemaphoreType.DMA,
    ],
    compiler_params=_sc_params,
)(_csum_body)


def _embed_body(nf0_ref, nf1_ref, a0_ref, a1_ref, out_ref):
    nf0 = nf0_ref[...]
    nf1 = nf1_ref[...]
    oh0 = (lax.broadcasted_iota(i32, (BR, 128), 1) == nf0).astype(f32)
    oh1 = (lax.broadcasted_iota(i32, (BR, 8), 1) == nf1).astype(f32)
    out_ref[...] = (jnp.dot(oh0, a0_ref[...], preferred_element_type=f32)
                    + jnp.dot(oh1, a1_ref[...], preferred_element_type=f32))


_embed = pl.pallas_call(
    _embed_body,
    grid=(N // BR,),
    in_specs=[
        pl.BlockSpec((BR, 1), lambda i: (i, 0)),
        pl.BlockSpec((BR, 1), lambda i: (i, 0)),
        pl.BlockSpec((128, DP), lambda i: (0, 0)),
        pl.BlockSpec((8, DP), lambda i: (0, 0)),
    ],
    out_specs=pl.BlockSpec((BR, DP), lambda i: (i, 0)),
    out_shape=jax.ShapeDtypeStruct((N, DP), f32),
)


def _mlp_body(relu_out, h_ref, agg_ref, c_ref, ee_ref, w1_ref, b1_ref,
              w2_ref, b2_ref, out_ref):
    z = (h_ref[...] + agg_ref[...]
         + jnp.dot(c_ref[...], ee_ref[...], preferred_element_type=f32))
    a = jnp.maximum(jnp.dot(z, w1_ref[...], preferred_element_type=f32)
                    + b1_ref[...], 0.0)
    y = jnp.dot(a, w2_ref[...], preferred_element_type=f32) + b2_ref[...]
    if relu_out:
        y = jnp.maximum(y, 0.0)
    out_ref[...] = y


def _make_mlp(relu_out):
    return pl.pallas_call(
        functools.partial(_mlp_body, relu_out),
        grid=(N // BR,),
        in_specs=[
            pl.BlockSpec((BR, DP), lambda i: (i, 0)),
            pl.BlockSpec((BR, DP), lambda i: (i, 0)),
            pl.BlockSpec((BR, 16), lambda i: (i, 0)),
            pl.BlockSpec((16, DP), lambda i: (0, 0)),
            pl.BlockSpec((DP, HP), lambda i: (0, 0)),
            pl.BlockSpec((1, HP), lambda i: (0, 0)),
            pl.BlockSpec((HP, DP), lambda i: (0, 0)),
            pl.BlockSpec((1, DP), lambda i: (0, 0)),
        ],
        out_specs=pl.BlockSpec((BR, DP), lambda i: (i, 0)),
        out_shape=jax.ShapeDtypeStruct((N, DP), f32),
    )


_mlp_mid = _make_mlp(True)
_mlp_last = _make_mlp(False)


def _pool_body(h_ref, gid_ref, out_ref, ssum, cnt):
    k = pl.program_id(0)

    @pl.when(k == 0)
    def _():
        ssum[...] = jnp.zeros_like(ssum)
        cnt[...] = jnp.zeros_like(cnt)

    gid = gid_ref[...]
    oh = (lax.broadcasted_iota(i32, (BR, G), 1) == gid).astype(f32)
    ssum[...] += lax.dot_general(oh, h_ref[...], (((0,), (0,)), ((), ())),
                                 preferred_element_type=f32)
    cnt[...] += lax.dot_general(oh, jnp.ones((BR, 128), f32),
                                (((0,), (0,)), ((), ())),
                                preferred_element_type=f32)

    c = cnt[:, 0:1]
    gh = jnp.where(c > 0, ssum[...] / jnp.maximum(c, 1.0), 0.0)
    out_ref[...] = jnp.maximum(gh[:, :D], 0.0)


_pool = pl.pallas_call(
    _pool_body,
    grid=(N // BR,),
    in_specs=[
        pl.BlockSpec((BR, DP), lambda i: (i, 0)),
        pl.BlockSpec((BR, 1), lambda i: (i, 0)),
    ],
    out_specs=pl.BlockSpec((G, D), lambda i: (0, 0)),
    out_shape=jax.ShapeDtypeStruct((G, D), f32),
    scratch_shapes=[pltpu.VMEM((G, DP), f32), pltpu.VMEM((G, 128), f32)],
)


def kernel(atom_emb0, atom_emb1, edge_emb0, edge_emb1, W1, b1, W2, b2,
           gamma, beta, rmean, rvar,
           node_feat0, node_feat1, edge_index, edge_feat0, edge_feat1,
           graph_ids):
    bn_s = gamma / jnp.sqrt(rvar + 1e-5)
    W2f = W2 * bn_s[:, None, :]
    b2f = b2 * bn_s + (beta - rmean * bn_s)

    W1p = jnp.zeros((L, DP, HP), f32).at[:, :D, :H].set(W1)
    b1p = jnp.zeros((L, 1, HP), f32).at[:, 0, :H].set(b1)
    W2p = jnp.zeros((L, HP, DP), f32).at[:, :H, :D].set(W2f)
    b2p = jnp.zeros((L, 1, DP), f32).at[:, 0, :D].set(b2f)
    EE = (jnp.zeros((L, 16, DP), f32)
          .at[:, 0:6, :D].set(edge_emb0)
          .at[:, 6:9, :D].set(edge_emb1))
    A0p = jnp.zeros((128, DP), f32).at[:120, :D].set(atom_emb0)
    A1p = jnp.zeros((8, DP), f32).at[:3, :D].set(atom_emb1)

    nf0 = node_feat0.astype(i32).reshape(N, 1)
    nf1 = node_feat1.astype(i32).reshape(N, 1)
    src = edge_index[0].astype(i32)
    dst = edge_index[1].astype(i32)
    pad = EP - E
    srcp = jnp.concatenate([src, jnp.zeros((pad,), i32)])
    dstp = jnp.concatenate([dst, jnp.full((pad,), N, i32)])
    f0p = jnp.concatenate([edge_feat0.astype(i32), jnp.zeros((pad,), i32)])
    f1p = jnp.concatenate([edge_feat1.astype(i32), jnp.zeros((pad,), i32)])
    gid = graph_ids.astype(i32).reshape(N, 1)
    Z = jnp.zeros((NH + 1, DP), f32)
    Z16 = jnp.zeros((NH + 1, 16), f32)

    r18 = jnp.arange(18)
    Tp = (jnp.zeros((24, 16), f32)
          .at[r18, r18 // 3].add(1.0)
          .at[r18, r18 % 3 + 6].add(1.0))

    h = _embed(nf0, nf1, A0p, A1p)
    csrc, cdstl, ccidx, ccnt = _compact(srcp, dstp, f0p, f1p)
    C = _csum(Tp, ccidx, cdstl, ccnt, Z16)
    for l in range(L):
        agg = _spmm(h, csrc, cdstl, ccnt, Z)
        mlp = _mlp_mid if l < L - 1 else _mlp_last
        h = mlp(h, agg, C, EE[l], W1p[l], b1p[l], W2p[l], b2p[l])
    return _pool(h, gid)
